# baseline (speedup 1.0000x reference)
import jax
import jax.numpy as jnp
from jax.experimental import pallas as pl


def _add_body(p_ref, x_ref, o_ref):
    o_ref[...] = x_ref[...] + p_ref[...][None]


def kernel(x, pos_table):
    B, S, D = x.shape
    BLK = 256
    BB = 4
    grid = (S // BLK, B // BB)
    return pl.pallas_call(
        _add_body,
        grid=grid,
        in_specs=[
            pl.BlockSpec((BLK, D), lambda s, b: (s, 0)),
            pl.BlockSpec((BB, BLK, D), lambda s, b: (b, s, 0)),
        ],
        out_specs=pl.BlockSpec((BB, BLK, D), lambda s, b: (b, s, 0)),
        out_shape=jax.ShapeDtypeStruct((B, S, D), x.dtype),
    )(pos_table, x)


# speedup vs baseline: 1.0140x; 1.0140x over previous
import jax
import jax.numpy as jnp
from jax.experimental import pallas as pl


def _add_body(p_ref, x_ref, o_ref):
    o_ref[...] = x_ref[...] + p_ref[...][None]


def kernel(x, pos_table):
    B, S, D = x.shape
    BLK = 1024
    BB = 2
    grid = (S // BLK, B // BB)
    return pl.pallas_call(
        _add_body,
        grid=grid,
        in_specs=[
            pl.BlockSpec((BLK, D), lambda s, b: (s, 0)),
            pl.BlockSpec((BB, BLK, D), lambda s, b: (b, s, 0)),
        ],
        out_specs=pl.BlockSpec((BB, BLK, D), lambda s, b: (b, s, 0)),
        out_shape=jax.ShapeDtypeStruct((B, S, D), x.dtype),
    )(pos_table, x)


# speedup vs baseline: 1.0143x; 1.0002x over previous
import jax
import jax.numpy as jnp
from jax.experimental import pallas as pl


def _add_body(p_ref, x_ref, o_ref):
    o_ref[...] = x_ref[...] + p_ref[...][None]


def kernel(x, pos_table):
    B, S, D = x.shape
    BLK = 2048
    BB = 1
    grid = (S // BLK, B // BB)
    return pl.pallas_call(
        _add_body,
        grid=grid,
        in_specs=[
            pl.BlockSpec((BLK, D), lambda s, b: (s, 0)),
            pl.BlockSpec((BB, BLK, D), lambda s, b: (b, s, 0)),
        ],
        out_specs=pl.BlockSpec((BB, BLK, D), lambda s, b: (b, s, 0)),
        out_shape=jax.ShapeDtypeStruct((B, S, D), x.dtype),
    )(pos_table, x)
